# Initial kernel scaffold; baseline (speedup 1.0000x reference)
import jax
import jax.numpy as jnp
from jax.experimental import pallas as pl


def kernel(hidden_states, gate_w, gate_up_weights, down_weights):
    raise NotImplementedError("write your pallas kernel here")


# speedup vs baseline: 1.3557x; 1.3557x over previous
import jax
import jax.numpy as jnp
from jax.experimental import pallas as pl
from jax.experimental.pallas import tpu as pltpu

_NE = 8
_D = 2048
_FF = 4096
_JITTER = 0.01
_BF = 512
_NFB = _FF // _BF


def _sparsemixer_weights(scores):
    neg_inf = jnp.float32(-jnp.inf)
    max_val = jnp.max(scores, axis=-1, keepdims=True)
    oh1 = scores >= max_val
    factor = jnp.maximum(jnp.abs(scores), max_val)
    mask1 = (max_val - scores) / factor > 2 * _JITTER
    masked_gates = jnp.where(mask1, neg_inf, scores)
    m1 = jnp.max(masked_gates, axis=-1, keepdims=True)
    e1 = jnp.exp(masked_gates - m1)
    p1 = e1 / jnp.sum(e1, axis=-1, keepdims=True)
    mult1 = jnp.sum(jnp.where(oh1, p1, 0.0), axis=-1, keepdims=True)

    masked_scores = jnp.where(oh1, neg_inf, scores)
    max_val2 = jnp.max(masked_scores, axis=-1, keepdims=True)
    oh2 = masked_scores >= max_val2
    factor2 = jnp.maximum(jnp.abs(scores), max_val2)
    mask2 = (max_val2 - scores) / factor2 > 2 * _JITTER
    masked_gates2 = jnp.where(mask2, neg_inf, masked_scores)
    m2 = jnp.max(masked_gates2, axis=-1, keepdims=True)
    e2 = jnp.exp(masked_gates2 - m2)
    p2 = e2 / jnp.sum(e2, axis=-1, keepdims=True)
    mult2 = jnp.sum(jnp.where(oh2, p2, 0.0), axis=-1, keepdims=True)

    return mult1 * oh1.astype(jnp.float32) + mult2 * oh2.astype(jnp.float32)


def _moe_kernel(x_ref, gw_ref, gup_g_ref, gup_u_ref, dn_ref,
                out_ref, logits_ref, w_sc, xb_sc):
    e = pl.program_id(0)
    fb = pl.program_id(1)

    @pl.when(jnp.logical_and(e == 0, fb == 0))
    def _router():
        x = x_ref[...]
        xb = x.astype(jnp.bfloat16)
        xb_sc[...] = xb
        logits = jax.lax.dot_general(
            xb, gw_ref[...].astype(jnp.bfloat16), (((1,), (1,)), ((), ())),
            preferred_element_type=jnp.float32)
        logits_ref[...] = logits
        w_sc[...] = _sparsemixer_weights(logits)

    xb = xb_sc[...]
    gb = gup_g_ref[0].astype(jnp.bfloat16)
    ub = gup_u_ref[0].astype(jnp.bfloat16)
    g = jax.lax.dot_general(xb, gb, (((1,), (1,)), ((), ())),
                            preferred_element_type=jnp.float32)
    u = jax.lax.dot_general(xb, ub, (((1,), (1,)), ((), ())),
                            preferred_element_type=jnp.float32)
    h = g * jax.nn.sigmoid(g) * u

    lane = jax.lax.broadcasted_iota(jnp.int32, (1, _NE), 1)
    wcol = jnp.sum(jnp.where(lane == e, w_sc[...], 0.0), axis=-1,
                   keepdims=True)
    hb = (h * wcol).astype(jnp.bfloat16)
    db = dn_ref[0].astype(jnp.bfloat16)
    y = jax.lax.dot_general(hb, db, (((1,), (1,)), ((), ())),
                            preferred_element_type=jnp.float32)

    @pl.when(jnp.logical_and(e == 0, fb == 0))
    def _init():
        out_ref[...] = y

    @pl.when(jnp.logical_or(e != 0, fb != 0))
    def _acc():
        out_ref[...] += y


def kernel(hidden_states, gate_w, gate_up_weights, down_weights):
    B, S, d = hidden_states.shape
    T = B * S
    x = hidden_states.reshape(T, d)

    out, logits = pl.pallas_call(
        _moe_kernel,
        grid=(_NE, _NFB),
        in_specs=[
            pl.BlockSpec((T, _D), lambda e, f: (0, 0)),
            pl.BlockSpec((_NE, _D), lambda e, f: (0, 0)),
            pl.BlockSpec((1, _BF, _D), lambda e, f: (e, f, 0)),
            pl.BlockSpec((1, _BF, _D), lambda e, f: (e, _NFB + f, 0)),
            pl.BlockSpec((1, _D, _BF), lambda e, f: (e, 0, f)),
        ],
        out_specs=[
            pl.BlockSpec((T, _D), lambda e, f: (0, 0)),
            pl.BlockSpec((T, _NE), lambda e, f: (0, 0)),
        ],
        out_shape=[
            jax.ShapeDtypeStruct((T, _D), jnp.float32),
            jax.ShapeDtypeStruct((T, _NE), jnp.float32),
        ],
        scratch_shapes=[
            pltpu.VMEM((T, _NE), jnp.float32),
            pltpu.VMEM((T, _D), jnp.bfloat16),
        ],
    )(x, gate_w, gate_up_weights, gate_up_weights, down_weights)

    return out.reshape(B, S, d), logits
